# baseline (speedup 1.0000x reference)
import functools

import jax
import jax.numpy as jnp
from jax import lax
from jax.experimental import pallas as pl
from jax.experimental.pallas import tpu as pltpu
from jax.experimental.pallas import tpu_sc as plsc

NUM_EXPERTS = 64
TOP_K = 2
HIDDEN = 2048
Z_LOSS_COEFF = 0.001
AUX_LOSS_COEFF = 0.01

BLOCK_T = 512
NSTREAM = 2
NEG_HUGE = -3.0e38


def _router_rows(logits):
    iota = jax.lax.broadcasted_iota(jnp.int32, logits.shape, 0)

    m1 = jnp.max(logits, axis=0, keepdims=True)
    idx1 = jnp.min(jnp.where(logits == m1, iota, NUM_EXPERTS),
                   axis=0, keepdims=True)
    masked = jnp.where(iota == idx1, NEG_HUGE, logits)
    m2 = jnp.max(masked, axis=0, keepdims=True)
    idx2 = jnp.min(jnp.where(masked == m2, iota, NUM_EXPERTS),
                   axis=0, keepdims=True)

    ex = jnp.exp(logits - m1)
    denom = jnp.sum(ex, axis=0, keepdims=True)
    recip = 1.0 / denom
    lse = m1 + jnp.log(denom)

    w1 = recip
    w2 = jnp.exp(m2 - m1) * recip

    wts_t = jnp.concatenate([w1, w2], axis=0)
    idx_t = jnp.concatenate([idx1, idx2], axis=0)

    agg_part = jnp.sum(ex * recip, axis=1, keepdims=True)
    z_part = jnp.sum(lse * lse, axis=1, keepdims=True)
    return wts_t, idx_t, agg_part, z_part


def _router_block(*refs, nblk, num_tokens):
    x_refs = refs[:NSTREAM]
    w_ref = refs[NSTREAM]
    wts_refs = refs[NSTREAM + 1:2 * NSTREAM + 1]
    idx_refs = refs[2 * NSTREAM + 1:3 * NSTREAM + 1]
    z_ref = refs[3 * NSTREAM + 1]
    aggout_ref = refs[3 * NSTREAM + 2]
    agg_ref, zacc_ref = refs[3 * NSTREAM + 3:]

    i = pl.program_id(0)
    w = w_ref[...]

    agg_acc = jnp.zeros((NUM_EXPERTS, 1), jnp.float32)
    z_acc = jnp.zeros((1, 1), jnp.float32)
    for x_ref, wts_ref, idx_ref in zip(x_refs, wts_refs, idx_refs):
        logits = jax.lax.dot_general(
            w, x_ref[...],
            dimension_numbers=(((1,), (1,)), ((), ())),
            preferred_element_type=jnp.float32,
        )
        wts_t, idx_t, agg_part, z_part = _router_rows(logits)
        wts_ref[...] = wts_t
        idx_ref[...] = idx_t
        agg_acc += agg_part
        z_acc += z_part

    @pl.when(i == 0)
    def _init():
        agg_ref[...] = agg_acc
        zacc_ref[...] = z_acc

    @pl.when(i > 0)
    def _accum():
        agg_ref[...] += agg_acc
        zacc_ref[...] += z_acc

    @pl.when(i == nblk - 1)
    def _finalize():
        z_ref[...] = zacc_ref[...] * (Z_LOSS_COEFF / num_tokens)
        aggout_ref[...] = agg_ref[...]


def _tc_router(xf, W):
    num_tokens = xf.shape[0]
    nblk = num_tokens // (BLOCK_T * NSTREAM)
    stream_rows = num_tokens // NSTREAM

    def x_map(s):
        return lambda i: (i + s * nblk, 0)

    in_specs = [pl.BlockSpec((BLOCK_T, HIDDEN), x_map(s))
                for s in range(NSTREAM)]
    in_specs.append(pl.BlockSpec((NUM_EXPERTS, HIDDEN), lambda i: (0, 0)))

    out_specs = (
        [pl.BlockSpec((TOP_K, BLOCK_T), lambda i: (0, i))
         for _ in range(2 * NSTREAM)]
        + [pl.BlockSpec((1, 1), lambda i: (0, 0)),
           pl.BlockSpec((NUM_EXPERTS, 1), lambda i: (0, 0))])
    out_shape = (
        [jax.ShapeDtypeStruct((TOP_K, stream_rows), jnp.float32)
         for _ in range(NSTREAM)]
        + [jax.ShapeDtypeStruct((TOP_K, stream_rows), jnp.int32)
           for _ in range(NSTREAM)]
        + [jax.ShapeDtypeStruct((1, 1), jnp.float32),
           jax.ShapeDtypeStruct((NUM_EXPERTS, 1), jnp.float32)])

    return pl.pallas_call(
        functools.partial(_router_block, nblk=nblk, num_tokens=num_tokens),
        grid=(nblk,),
        in_specs=in_specs,
        out_specs=out_specs,
        out_shape=out_shape,
        scratch_shapes=[
            pltpu.VMEM((NUM_EXPERTS, 1), jnp.float32),
            pltpu.VMEM((1, 1), jnp.float32),
        ],
    )(*([xf] * NSTREAM), W)


NSUB = 16
IDX_PER_W = 1024


def _sc_hist_body(idx0_hbm, idx1_hbm, out_hbm,
                  idx_v, ones_v, tmp_v, hist_v, bins_sh):
    wid = lax.axis_index("s")
    half = IDX_PER_W // 2
    base = wid * half

    pltpu.sync_copy(idx0_hbm.at[pl.ds(base, half)],
                    idx_v.at[pl.ds(0, half)])
    pltpu.sync_copy(idx1_hbm.at[pl.ds(base, half)],
                    idx_v.at[pl.ds(half, half)])

    for j in range(IDX_PER_W // 16):
        ones_v[pl.ds(j * 16, 16)] = jnp.full((16,), 1.0, jnp.float32)

    @pl.when(wid == 0)
    def _init_bins():
        for j in range(NUM_EXPERTS // 16):
            tmp_v[pl.ds(j * 16, 16)] = jnp.zeros((16,), jnp.float32)
        pltpu.sync_copy(tmp_v, bins_sh)

    plsc.subcore_barrier()
    pltpu.sync_copy(ones_v, bins_sh.at[idx_v], add=True)
    plsc.subcore_barrier()

    @pl.when(wid == 0)
    def _finalize():
        pltpu.sync_copy(bins_sh, hist_v)
        pltpu.sync_copy(hist_v, out_hbm)


def _sc_hist(idx0_flat, idx1_flat):
    mesh = plsc.VectorSubcoreMesh(
        core_axis_name="c", subcore_axis_name="s",
        num_cores=1, num_subcores=NSUB)
    fn = pl.kernel(
        _sc_hist_body,
        out_type=jax.ShapeDtypeStruct((NUM_EXPERTS,), jnp.float32),
        mesh=mesh,
        scratch_types=[
            pltpu.VMEM((IDX_PER_W,), jnp.int32),
            pltpu.VMEM((IDX_PER_W,), jnp.float32),
            pltpu.VMEM((NUM_EXPERTS,), jnp.float32),
            pltpu.VMEM((NUM_EXPERTS,), jnp.float32),
            pltpu.MemorySpace.VMEM_SHARED((NUM_EXPERTS,), jnp.float32),
        ],
    )
    return fn(idx0_flat, idx1_flat)


def _dot_body(h_ref, a_ref, o_ref, *, num_tokens):
    o_ref[...] = jnp.sum(h_ref[...] * a_ref[...], keepdims=True) * (
        NUM_EXPERTS * AUX_LOSS_COEFF / (num_tokens * num_tokens * TOP_K))


def _tc_lbl(hist, agg, num_tokens):
    return pl.pallas_call(
        functools.partial(_dot_body, num_tokens=num_tokens),
        out_shape=jax.ShapeDtypeStruct((1, 1), jnp.float32),
    )(hist.reshape(1, NUM_EXPERTS), agg.reshape(1, NUM_EXPERTS))


def kernel(x, W):
    xf = x.reshape(-1, x.shape[-1])
    num_tokens = xf.shape[0]

    outs = _tc_router(xf, W)
    wts = jnp.concatenate(outs[:NSTREAM], axis=1).T
    idx = jnp.concatenate(outs[NSTREAM:2 * NSTREAM], axis=1).T
    z = outs[2 * NSTREAM][0, 0]
    agg = outs[2 * NSTREAM + 1].reshape(NUM_EXPERTS)

    idx0_flat = outs[NSTREAM].reshape(-1)
    idx1_flat = outs[NSTREAM + 1].reshape(-1)
    hist = _sc_hist(idx0_flat, idx1_flat)
    lbl = _tc_lbl(hist, agg, num_tokens)[0, 0]
    return wts, idx, z, lbl


# speedup vs baseline: 1.1761x; 1.1761x over previous
import functools

import jax
import jax.numpy as jnp
from jax.experimental import pallas as pl
from jax.experimental.pallas import tpu as pltpu

NUM_EXPERTS = 64
TOP_K = 2
HIDDEN = 2048
Z_LOSS_COEFF = 0.001
AUX_LOSS_COEFF = 0.01

BLOCK_T = 512
NSTREAM = 2
NEG_HUGE = -3.0e38


def _router_rows(logits):
    iota = jax.lax.broadcasted_iota(jnp.int32, logits.shape, 0)

    m1 = jnp.max(logits, axis=0, keepdims=True)
    idx1 = jnp.min(jnp.where(logits == m1, iota, NUM_EXPERTS),
                   axis=0, keepdims=True)
    masked = jnp.where(iota == idx1, NEG_HUGE, logits)
    m2 = jnp.max(masked, axis=0, keepdims=True)
    idx2 = jnp.min(jnp.where(masked == m2, iota, NUM_EXPERTS),
                   axis=0, keepdims=True)

    ex = jnp.exp(logits - m1)
    denom = jnp.sum(ex, axis=0, keepdims=True)
    recip = 1.0 / denom
    lse = m1 + jnp.log(denom)

    w1 = recip
    w2 = jnp.exp(m2 - m1) * recip

    wts_t = jnp.concatenate([w1, w2], axis=0)
    idx_t = jnp.concatenate([idx1, idx2], axis=0)

    onehot = ((iota == idx1) | (iota == idx2)).astype(jnp.float32)
    hist_part = jnp.sum(onehot, axis=1, keepdims=True)
    agg_part = jnp.sum(ex * recip, axis=1, keepdims=True)
    z_part = jnp.sum(lse * lse, axis=1, keepdims=True)
    return wts_t, idx_t, hist_part, agg_part, z_part


def _router_block(*refs, nblk, num_tokens):
    x_refs = refs[:NSTREAM]
    w_ref = refs[NSTREAM]
    wts_refs = refs[NSTREAM + 1:2 * NSTREAM + 1]
    idx_refs = refs[2 * NSTREAM + 1:3 * NSTREAM + 1]
    z_ref = refs[3 * NSTREAM + 1]
    lbl_ref = refs[3 * NSTREAM + 2]
    agg_ref, hist_ref, zacc_ref = refs[3 * NSTREAM + 3:]

    i = pl.program_id(0)
    w = w_ref[...]

    hist_acc = jnp.zeros((NUM_EXPERTS, 1), jnp.float32)
    agg_acc = jnp.zeros((NUM_EXPERTS, 1), jnp.float32)
    z_acc = jnp.zeros((1, 1), jnp.float32)
    for x_ref, wts_ref, idx_ref in zip(x_refs, wts_refs, idx_refs):
        logits = jax.lax.dot_general(
            w, x_ref[...],
            dimension_numbers=(((1,), (1,)), ((), ())),
            preferred_element_type=jnp.float32,
        )
        wts_t, idx_t, hist_part, agg_part, z_part = _router_rows(logits)
        wts_ref[...] = wts_t
        idx_ref[...] = idx_t
        hist_acc += hist_part
        agg_acc += agg_part
        z_acc += z_part

    @pl.when(i == 0)
    def _init():
        agg_ref[...] = agg_acc
        hist_ref[...] = hist_acc
        zacc_ref[...] = z_acc

    @pl.when(i > 0)
    def _accum():
        agg_ref[...] += agg_acc
        hist_ref[...] += hist_acc
        zacc_ref[...] += z_acc

    @pl.when(i == nblk - 1)
    def _finalize():
        z_ref[...] = zacc_ref[...] * (Z_LOSS_COEFF / num_tokens)
        lbl_ref[...] = jnp.sum(agg_ref[...] * hist_ref[...], keepdims=True) * (
            NUM_EXPERTS * AUX_LOSS_COEFF / (num_tokens * num_tokens * TOP_K))


def kernel(x, W):
    xf = x.reshape(-1, x.shape[-1])
    num_tokens = xf.shape[0]
    nblk = num_tokens // (BLOCK_T * NSTREAM)
    stream_rows = num_tokens // NSTREAM

    def x_map(s):
        return lambda i: (i + s * nblk, 0)

    in_specs = [pl.BlockSpec((BLOCK_T, HIDDEN), x_map(s))
                for s in range(NSTREAM)]
    in_specs.append(pl.BlockSpec((NUM_EXPERTS, HIDDEN), lambda i: (0, 0)))

    out_specs = (
        [pl.BlockSpec((TOP_K, BLOCK_T), lambda i: (0, i))
         for _ in range(2 * NSTREAM)]
        + [pl.BlockSpec((1, 1), lambda i: (0, 0))] * 2)
    out_shape = (
        [jax.ShapeDtypeStruct((TOP_K, stream_rows), jnp.float32)
         for _ in range(NSTREAM)]
        + [jax.ShapeDtypeStruct((TOP_K, stream_rows), jnp.int32)
           for _ in range(NSTREAM)]
        + [jax.ShapeDtypeStruct((1, 1), jnp.float32)] * 2)

    outs = pl.pallas_call(
        functools.partial(_router_block, nblk=nblk, num_tokens=num_tokens),
        grid=(nblk,),
        in_specs=in_specs,
        out_specs=out_specs,
        out_shape=out_shape,
        scratch_shapes=[
            pltpu.VMEM((NUM_EXPERTS, 1), jnp.float32),
            pltpu.VMEM((NUM_EXPERTS, 1), jnp.float32),
            pltpu.VMEM((1, 1), jnp.float32),
        ],
    )(*([xf] * NSTREAM), W)

    wts = jnp.concatenate(outs[:NSTREAM], axis=1).T
    idx = jnp.concatenate(outs[NSTREAM:2 * NSTREAM], axis=1).T
    z, lbl = outs[2 * NSTREAM], outs[2 * NSTREAM + 1]
    return wts, idx, z[0, 0], lbl[0, 0]
